# baseline (speedup 1.0000x reference)
import functools

import jax
import jax.numpy as jnp
from jax import lax
from jax.experimental import pallas as pl
from jax.experimental.pallas import tpu as pltpu
from jax.experimental.pallas import tpu_sc as plsc

N = 10000
E = 320000
H = 128
ED = 16

NC = 2
NS = 16
NW = NC * NS
C = 64
NCHUNK = E // C
KPT = (NCHUNK + NW - 1) // NW
RPS = 624
TAIL = N - NS * RPS

RC = 64
NRCH = E // RC
KRT = (NRCH + NW - 1) // NW


def _repack_body(ea_hbm, out_hbm, ea0, ob0, ea1, ob1, sem0, sem1):
    c = lax.axis_index("c")
    s = lax.axis_index("s")
    wid = s * NC + c
    bufs = ((ea0, ob0, sem0), (ea1, ob1, sem1))

    def _valid(k):
        return (wid + k * NW) < NRCH

    def _base(k):
        return (wid + k * NW) * RC

    def _issue(k, b):
        pltpu.async_copy(ea_hbm.at[pl.ds(_base(k), RC)], bufs[b][0],
                         bufs[b][2])

    def _do(k, b):
        ea, ob = bufs[b][0], bufs[b][1]
        pltpu.make_async_copy(ea_hbm.at[pl.ds(0, RC)], ea, bufs[b][2]).wait()

        def _row(j, _):
            for r in range(8):
                ob[j, pl.ds(r * ED, ED)] = ea[8 * j + r, :]
            return 0
        lax.fori_loop(0, RC // 8, _row, 0)
        pltpu.sync_copy(
            ob,
            out_hbm.at[pl.ds(pl.multiple_of(_base(k) // 8, RC // 8), RC // 8)])

    @pl.when(_valid(0))
    def _p0():
        _issue(0, 0)

    @pl.when(_valid(1))
    def _p1():
        _issue(1, 1)

    def _pair(p, _):
        for half in range(2):
            k = 2 * p + half
            cur = half

            @pl.when(_valid(k))
            def _w():
                _do(k, cur)

            @pl.when(_valid(k + 2))
            def _n():
                _issue(k + 2, cur)
        return 0

    lax.fori_loop(0, (KRT + 1) // 2, _pair, 0)


@functools.partial(
    pl.kernel,
    out_type=jax.ShapeDtypeStruct((E // 8, H), jnp.float32),
    mesh=plsc.VectorSubcoreMesh(core_axis_name="c", subcore_axis_name="s"),
    scratch_types=[
        pltpu.VMEM((RC, ED), jnp.float32),
        pltpu.VMEM((RC // 8, H), jnp.float32),
        pltpu.VMEM((RC, ED), jnp.float32),
        pltpu.VMEM((RC // 8, H), jnp.float32),
        pltpu.SemaphoreType.DMA,
        pltpu.SemaphoreType.DMA,
    ],
)
def _repack(ea_hbm, out_hbm, *scratch):
    _repack_body(ea_hbm, out_hbm, *scratch)


def _proj_body(ea_ref, wt_ref, b_ref, out_ref):
    for r in range(8):
        out_ref[r] = (
            jnp.dot(ea_ref[:, r * ED:(r + 1) * ED], wt_ref[...],
                    preferred_element_type=jnp.float32)
            + b_ref[...]
        )


def _edge_proj(ea8, w_t, b_row):
    BE = 400
    grid = (E // 8) // BE
    return pl.pallas_call(
        _proj_body,
        grid=(grid,),
        in_specs=[
            pl.BlockSpec((BE, H), lambda i: (i, 0)),
            pl.BlockSpec((ED, H), lambda i: (0, 0)),
            pl.BlockSpec((1, H), lambda i: (0, 0)),
        ],
        out_specs=pl.BlockSpec((8, BE, H), lambda i: (0, i, 0)),
        out_shape=jax.ShapeDtypeStruct((8, E // 8, H), jnp.float32),
    )(ea8, w_t, b_row)


def _scatter_body(x_hbm, src_hbm, dst_hbm, proj_hbm, out_hbm,
                  acc, srcv0, dstv0, xbuf0, pbuf0, srcv1, dstv1, xbuf1, pbuf1,
                  sem_i0, sem_g0, sem_p0, sem_i1, sem_g1, sem_p1):
    c = lax.axis_index("c")
    s = lax.axis_index("s")
    wid = s * NC + c
    bufs = ((srcv0, dstv0, xbuf0, pbuf0, sem_i0, sem_g0, sem_p0),
            (srcv1, dstv1, xbuf1, pbuf1, sem_i1, sem_g1, sem_p1))

    def _zrow(i, _):
        for h in range(H // 16):
            xbuf0[i, pl.ds(h * 16, 16)] = jnp.zeros((16,), jnp.float32)
        return 0
    lax.fori_loop(0, C, _zrow, 0)
    for k in range(9):
        pltpu.sync_copy(xbuf0, acc.at[pl.ds(s * RPS + k * C, C)])
    pltpu.sync_copy(xbuf0.at[pl.ds(0, RPS - 9 * C)],
                    acc.at[pl.ds(s * RPS + 9 * C, RPS - 9 * C)])

    @pl.when(s == 0)
    def _zero_tail():
        pltpu.sync_copy(xbuf0.at[pl.ds(0, TAIL)], acc.at[pl.ds(NS * RPS, TAIL)])

    plsc.subcore_barrier()

    def _valid(k):
        return (wid + k * NW) < NCHUNK

    def _base(k):
        return (wid + k * NW) * C

    def _issue_idx(k, b):
        srcv, dstv = bufs[b][0], bufs[b][1]
        sem = bufs[b][4]
        pltpu.async_copy(src_hbm.at[pl.ds(_base(k), C)], srcv, sem)
        pltpu.async_copy(dst_hbm.at[pl.ds(_base(k), C)], dstv, sem)

    def _wait_idx(b):
        srcv, dstv, sem = bufs[b][0], bufs[b][1], bufs[b][4]
        pltpu.make_async_copy(src_hbm.at[pl.ds(0, C)], srcv, sem).wait()
        pltpu.make_async_copy(dst_hbm.at[pl.ds(0, C)], dstv, sem).wait()

    def _issue_fetch(k, b):
        srcv, xbuf, pbuf = bufs[b][0], bufs[b][2], bufs[b][3]
        pltpu.async_copy(
            proj_hbm.at[:, pl.ds(pl.multiple_of((wid + k * NW) * (C // 8),
                                                C // 8), C // 8), :],
            pbuf, bufs[b][6])
        pltpu.async_copy(x_hbm.at[srcv], xbuf, bufs[b][5])

    def _compute_scatter(b):
        srcv, dstv, xbuf, pbuf = bufs[b][:4]
        pltpu.make_async_copy(
            proj_hbm.at[:, pl.ds(0, C // 8), :], pbuf, bufs[b][6]).wait()
        pltpu.make_async_copy(x_hbm.at[srcv], xbuf, bufs[b][5]).wait()

        def _row(j, _):
            for r in range(8):
                i = 8 * j + r
                for h in range(H // 16):
                    sl = pl.ds(h * 16, 16)
                    xbuf[i, sl] = jnp.maximum(xbuf[i, sl] + pbuf[r, j, sl], 0.0)
            return 0
        lax.fori_loop(0, C // 8, _row, 0)
        pltpu.sync_copy(xbuf, acc.at[dstv], add=True)

    _issue_idx(0, 0)
    _wait_idx(0)
    _issue_fetch(0, 0)
    _issue_idx(1, 1)

    def _pair(p, _):
        for half in range(2):
            k = 2 * p + half
            cur = half
            nxt = 1 - cur

            @pl.when(_valid(k + 1))
            def _pf():
                _wait_idx(nxt)
                _issue_fetch(k + 1, nxt)

            @pl.when(_valid(k))
            def _do():
                _compute_scatter(cur)

            @pl.when(_valid(k + 2))
            def _pi():
                _issue_idx(k + 2, cur)
        return 0

    lax.fori_loop(0, (KPT + 1) // 2, _pair, 0)
    plsc.subcore_barrier()

    pltpu.sync_copy(
        acc.at[pl.ds(s * RPS, RPS)],
        out_hbm.at[c, pl.ds(s * RPS, RPS)],
    )

    @pl.when(s == 0)
    def _write_tail():
        pltpu.sync_copy(
            acc.at[pl.ds(NS * RPS, TAIL)],
            out_hbm.at[c, pl.ds(NS * RPS, TAIL)],
        )


@functools.partial(
    pl.kernel,
    out_type=jax.ShapeDtypeStruct((NC, N, H), jnp.float32),
    mesh=plsc.VectorSubcoreMesh(core_axis_name="c", subcore_axis_name="s"),
    scratch_types=[
        pltpu.VMEM_SHARED((N, H), jnp.float32),
        pltpu.VMEM((C,), jnp.int32),
        pltpu.VMEM((C,), jnp.int32),
        pltpu.VMEM((C, H), jnp.float32),
        pltpu.VMEM((8, C // 8, H), jnp.float32),
        pltpu.VMEM((C,), jnp.int32),
        pltpu.VMEM((C,), jnp.int32),
        pltpu.VMEM((C, H), jnp.float32),
        pltpu.VMEM((8, C // 8, H), jnp.float32),
        pltpu.SemaphoreType.DMA,
        pltpu.SemaphoreType.DMA,
        pltpu.SemaphoreType.DMA,
        pltpu.SemaphoreType.DMA,
        pltpu.SemaphoreType.DMA,
        pltpu.SemaphoreType.DMA,
    ],
)
def _scatter(x_hbm, src_hbm, dst_hbm, proj_hbm, out_hbm, *scratch):
    _scatter_body(x_hbm, src_hbm, dst_hbm, proj_hbm, out_hbm, *scratch)


def _mlp_body(eps_ref, x_ref, parts_ref, w1t_ref, b1_ref, g1_ref, bt1_ref,
              w2t_ref, b2_ref, go_ref, bo_ref, out_ref):
    def _sigmoid(v):
        return 1.0 / (1.0 + jnp.exp(-v))

    def _bn(z, g, b):
        m = jnp.mean(z, axis=0, keepdims=True)
        v = jnp.mean((z - m) * (z - m), axis=0, keepdims=True)
        return (z - m) * jax.lax.rsqrt(v + 1e-5) * g + b

    agg = parts_ref[0] + parts_ref[1]
    h0 = (1.0 + eps_ref[0]) * x_ref[...] + agg
    z1 = jnp.dot(h0, w1t_ref[...], preferred_element_type=jnp.float32) + b1_ref[...]
    bn1 = _bn(z1, g1_ref[...], bt1_ref[...])
    a1 = bn1 * _sigmoid(bn1)
    z2 = jnp.dot(a1, w2t_ref[...], preferred_element_type=jnp.float32) + b2_ref[...]
    h = _bn(z2, go_ref[...], bo_ref[...])
    out_ref[...] = h * _sigmoid(h)


def _mlp(eps_1, x, parts, w1t, b1, g1, bt1, w2t, b2, go, bo):
    vspec = pl.BlockSpec(memory_space=pltpu.MemorySpace.VMEM)
    return pl.pallas_call(
        _mlp_body,
        in_specs=[pl.BlockSpec(memory_space=pltpu.MemorySpace.SMEM)]
        + [vspec] * 10,
        out_specs=vspec,
        out_shape=jax.ShapeDtypeStruct((N, H), jnp.float32),
    )(eps_1, x, parts, w1t, b1, g1, bt1, w2t, b2, go, bo)


def kernel(x, edge_index, edge_attr, W_e, b_e, eps, W1, b1, g1, beta1,
           W2, b2, g_out, beta_out):
    src = edge_index[0]
    dst = edge_index[1]
    ea8 = _repack(edge_attr)
    proj = _edge_proj(ea8, W_e.T, b_e.reshape(1, H))
    parts = _scatter(x, src, dst, proj)
    return _mlp(
        eps.reshape(1), x, parts,
        W1.T, b1.reshape(1, 2 * H), g1.reshape(1, 2 * H),
        beta1.reshape(1, 2 * H),
        W2.T, b2.reshape(1, H), g_out.reshape(1, H), beta_out.reshape(1, H),
    )


# speedup vs baseline: 1.8069x; 1.8069x over previous
import functools

import jax
import jax.numpy as jnp
from jax import lax
from jax.experimental import pallas as pl
from jax.experimental.pallas import tpu as pltpu
from jax.experimental.pallas import tpu_sc as plsc

N = 10000
E = 320000
H = 128
ED = 16

NC = 2
NS = 16
NW = NC * NS
C = 80
NCHUNK = E // C
KPT = NCHUNK // NW
RPS = 624
TAIL = N - NS * RPS


def _proj_body(ea_ref, wt_ref, b_ref, out_ref):
    out_ref[...] = (
        jnp.dot(ea_ref[...], wt_ref[...], preferred_element_type=jnp.float32)
        + b_ref[...]
    )


def _edge_proj(edge_attr, w_t, b_row):
    BE = 3200
    grid = E // BE
    return pl.pallas_call(
        _proj_body,
        grid=(grid,),
        in_specs=[
            pl.BlockSpec((BE, ED), lambda i: (i, 0)),
            pl.BlockSpec((ED, H), lambda i: (0, 0)),
            pl.BlockSpec((1, H), lambda i: (0, 0)),
        ],
        out_specs=pl.BlockSpec((BE, H), lambda i: (i, 0)),
        out_shape=jax.ShapeDtypeStruct((E, H), jnp.float32),
    )(edge_attr, w_t, b_row)


def _scatter_body(x_hbm, src_hbm, dst_hbm, proj_hbm, out_hbm,
                  acc, srcv0, dstv0, xbuf0, pbuf0, srcv1, dstv1, xbuf1, pbuf1,
                  sem_i0, sem_g0, sem_p0, sem_i1, sem_g1, sem_p1):
    c = lax.axis_index("c")
    s = lax.axis_index("s")
    wid = s * NC + c
    bufs = ((srcv0, dstv0, xbuf0, pbuf0, sem_i0, sem_g0, sem_p0),
            (srcv1, dstv1, xbuf1, pbuf1, sem_i1, sem_g1, sem_p1))

    def _zrow(i, _):
        for h in range(H // 16):
            xbuf0[i, pl.ds(h * 16, 16)] = jnp.zeros((16,), jnp.float32)
        return 0
    lax.fori_loop(0, C, _zrow, 0)
    for k in range(7):
        pltpu.sync_copy(xbuf0, acc.at[pl.ds(s * RPS + k * C, C)])
    pltpu.sync_copy(xbuf0.at[pl.ds(0, RPS - 7 * C)],
                    acc.at[pl.ds(s * RPS + 7 * C, RPS - 7 * C)])

    @pl.when(s == 0)
    def _zero_tail():
        pltpu.sync_copy(xbuf0.at[pl.ds(0, TAIL)], acc.at[pl.ds(NS * RPS, TAIL)])

    plsc.subcore_barrier()

    def _base(k):
        return (wid + k * NW) * C

    def _issue_idx(k, b):
        srcv, dstv = bufs[b][0], bufs[b][1]
        sem = bufs[b][4]
        pltpu.async_copy(src_hbm.at[pl.ds(_base(k), C)], srcv, sem)
        pltpu.async_copy(dst_hbm.at[pl.ds(_base(k), C)], dstv, sem)

    def _wait_idx(b):
        srcv, dstv, sem = bufs[b][0], bufs[b][1], bufs[b][4]
        pltpu.make_async_copy(src_hbm.at[pl.ds(0, C)], srcv, sem).wait()
        pltpu.make_async_copy(dst_hbm.at[pl.ds(0, C)], dstv, sem).wait()

    def _issue_fetch(k, b):
        srcv, xbuf, pbuf = bufs[b][0], bufs[b][2], bufs[b][3]
        pltpu.async_copy(proj_hbm.at[pl.ds(_base(k), C)], pbuf, bufs[b][6])
        pltpu.async_copy(x_hbm.at[srcv], xbuf, bufs[b][5])

    def _compute_scatter(b):
        srcv, dstv, xbuf, pbuf = bufs[b][:4]
        pltpu.make_async_copy(proj_hbm.at[pl.ds(0, C)], pbuf, bufs[b][6]).wait()
        pltpu.make_async_copy(x_hbm.at[srcv], xbuf, bufs[b][5]).wait()

        def _row(i, _):
            for h in range(H // 16):
                sl = pl.ds(h * 16, 16)
                xbuf[i, sl] = jnp.maximum(xbuf[i, sl] + pbuf[i, sl], 0.0)
            return 0
        lax.fori_loop(0, C, _row, 0)
        pltpu.sync_copy(xbuf, acc.at[dstv], add=True)

    _issue_idx(0, 0)
    _wait_idx(0)
    _issue_fetch(0, 0)
    _issue_idx(1, 1)

    def _pair(p, _):
        for half in range(2):
            k = 2 * p + half
            cur = half
            nxt = 1 - cur

            @pl.when(k + 1 < KPT)
            def _pf():
                _wait_idx(nxt)
                _issue_fetch(k + 1, nxt)

            @pl.when(k < KPT)
            def _do():
                _compute_scatter(cur)

            @pl.when(k + 2 < KPT)
            def _pi():
                _issue_idx(k + 2, cur)
        return 0

    lax.fori_loop(0, (KPT + 1) // 2, _pair, 0)
    plsc.subcore_barrier()

    pltpu.sync_copy(
        acc.at[pl.ds(s * RPS, RPS)],
        out_hbm.at[c, pl.ds(s * RPS, RPS)],
    )

    @pl.when(s == 0)
    def _write_tail():
        pltpu.sync_copy(
            acc.at[pl.ds(NS * RPS, TAIL)],
            out_hbm.at[c, pl.ds(NS * RPS, TAIL)],
        )


@functools.partial(
    pl.kernel,
    out_type=jax.ShapeDtypeStruct((NC, N, H), jnp.float32),
    mesh=plsc.VectorSubcoreMesh(core_axis_name="c", subcore_axis_name="s"),
    scratch_types=[
        pltpu.VMEM_SHARED((N, H), jnp.float32),
        pltpu.VMEM((C,), jnp.int32),
        pltpu.VMEM((C,), jnp.int32),
        pltpu.VMEM((C, H), jnp.float32),
        pltpu.VMEM((C, H), jnp.float32),
        pltpu.VMEM((C,), jnp.int32),
        pltpu.VMEM((C,), jnp.int32),
        pltpu.VMEM((C, H), jnp.float32),
        pltpu.VMEM((C, H), jnp.float32),
        pltpu.SemaphoreType.DMA,
        pltpu.SemaphoreType.DMA,
        pltpu.SemaphoreType.DMA,
        pltpu.SemaphoreType.DMA,
        pltpu.SemaphoreType.DMA,
        pltpu.SemaphoreType.DMA,
    ],
)
def _scatter(x_hbm, src_hbm, dst_hbm, proj_hbm, out_hbm, *scratch):
    _scatter_body(x_hbm, src_hbm, dst_hbm, proj_hbm, out_hbm, *scratch)


def _mlp_body(eps_ref, x_ref, parts_ref, w1t_ref, b1_ref, g1_ref, bt1_ref,
              w2t_ref, b2_ref, go_ref, bo_ref, out_ref):
    def _sigmoid(v):
        return 1.0 / (1.0 + jnp.exp(-v))

    def _bn(z, g, b):
        m = jnp.mean(z, axis=0, keepdims=True)
        v = jnp.mean((z - m) * (z - m), axis=0, keepdims=True)
        return (z - m) * jax.lax.rsqrt(v + 1e-5) * g + b

    agg = parts_ref[0] + parts_ref[1]
    h0 = (1.0 + eps_ref[0]) * x_ref[...] + agg
    z1 = jnp.dot(h0, w1t_ref[...], preferred_element_type=jnp.float32) + b1_ref[...]
    bn1 = _bn(z1, g1_ref[...], bt1_ref[...])
    a1 = bn1 * _sigmoid(bn1)
    z2 = jnp.dot(a1, w2t_ref[...], preferred_element_type=jnp.float32) + b2_ref[...]
    h = _bn(z2, go_ref[...], bo_ref[...])
    out_ref[...] = h * _sigmoid(h)


def _mlp(eps_1, x, parts, w1t, b1, g1, bt1, w2t, b2, go, bo):
    vspec = pl.BlockSpec(memory_space=pltpu.MemorySpace.VMEM)
    return pl.pallas_call(
        _mlp_body,
        in_specs=[pl.BlockSpec(memory_space=pltpu.MemorySpace.SMEM)]
        + [vspec] * 10,
        out_specs=vspec,
        out_shape=jax.ShapeDtypeStruct((N, H), jnp.float32),
    )(eps_1, x, parts, w1t, b1, g1, bt1, w2t, b2, go, bo)


def kernel(x, edge_index, edge_attr, W_e, b_e, eps, W1, b1, g1, beta1,
           W2, b2, g_out, beta_out):
    src = edge_index[0]
    dst = edge_index[1]
    proj = _edge_proj(edge_attr, W_e.T, b_e.reshape(1, H))
    parts = _scatter(x, src, dst, proj)
    return _mlp(
        eps.reshape(1), x, parts,
        W1.T, b1.reshape(1, 2 * H), g1.reshape(1, 2 * H),
        beta1.reshape(1, 2 * H),
        W2.T, b2.reshape(1, H), g_out.reshape(1, H), beta_out.reshape(1, H),
    )
